# baseline (speedup 1.0000x reference)
import functools

import jax
import jax.numpy as jnp
from jax import lax
from jax.experimental import pallas as pl
from jax.experimental.pallas import tpu as pltpu
from jax.experimental.pallas import tpu_sc as plsc

F = 26
V = 100000
VH = 50048
VR = V - VH
D = 32
B = 16384
L = 16

NC, NS = 2, 16

CQ = 4096
NQ = B // CQ

VB = VR - 32
_HLEN = (VH, VR)
_HOFF = (0, VH)

_mesh = plsc.VectorSubcoreMesh(core_axis_name="c", subcore_axis_name="s")


@functools.partial(
    pl.kernel,
    mesh=_mesh,
    out_type=jax.ShapeDtypeStruct((D, B), jnp.float32),
    scratch_types=[
        pltpu.VMEM((VH,), jnp.float32),
        pltpu.VMEM((VH,), jnp.float32),
        pltpu.VMEM((CQ,), jnp.int32),
        pltpu.VMEM((CQ,), jnp.int32),
        pltpu.VMEM((B,), jnp.float32),
        pltpu.SemaphoreType.DMA,
        pltpu.SemaphoreType.DMA,
        pltpu.SemaphoreType.DMA,
        pltpu.SemaphoreType.DMA,
    ],
    compiler_params=pltpu.CompilerParams(needs_layout_passes=False),
)
def _emb_kernel(ct_hbm, tt_hbm, out_hbm, rowa_v, rowb_v, cq0_v, cq1_v,
                acc_v, sem_ra, sem_rb, sem_c0, sem_c1):
    d = lax.axis_index("s") * NC + lax.axis_index("c")
    rbufs = (rowa_v, rowb_v)
    rsems = (sem_ra, sem_rb)
    cbufs = (cq0_v, cq1_v)
    csems = (sem_c0, sem_c1)

    def _half_copies(f, h, make):
        ctor = pltpu.make_async_copy if make else pltpu.async_copy
        if h == 0:
            return [ctor(tt_hbm.at[f, d, pl.ds(0, VH)], rowa_v, sem_ra)]
        return [
            ctor(tt_hbm.at[f, d, pl.ds(VH, VB)],
                 rowb_v.at[pl.ds(0, VB)], sem_rb),
            ctor(tl_hbm.at[pl.ds(f * (D * 32) + d * 32, 32)],
                 rowb_v.at[pl.ds(VB, 32)], sem_rb),
        ]

    def start_half(f, h):
        _half_copies(f, h, make=False)

    def wait_half(f, h):
        for cp in _half_copies(f, h, make=True):
            cp.wait()

    def half_pass(f, h, store):
        wait_half(f, h)
        cps = [
            pltpu.async_copy(ct_hbm.at[f, pl.ds(0, CQ)], cbufs[0],
                             csems[0]),
            None,
        ]
        for q in range(NQ):
            b = q % 2
            cps[b].wait()
            if q + 1 < NQ:
                nb = (q + 1) % 2
                cps[nb] = pltpu.async_copy(
                    ct_hbm.at[f, pl.ds((q + 1) * CQ, CQ)], cbufs[nb],
                    csems[nb])
            cq_v = cbufs[b]
            lo = jnp.int32(_HOFF[h])
            hi = jnp.int32(_HLEN[h])

            def k_body(k, carry3):
                idx = cq_v[pl.ds(k * L, L)]
                local = idx - lo
                m = (local >= 0) & (local < hi)
                g = plsc.load_gather(rbufs[h], [local], mask=m)
                g = jnp.where(m, g, jnp.float32(0.0))
                sl = pl.ds(q * CQ + k * L, L)
                if store:
                    acc_v[sl] = g
                else:
                    plsc.addupdate(acc_v.at[sl], g)
                return carry3

            lax.fori_loop(0, CQ // L, k_body, 0, unroll=16)

    start_half(0, 0)
    start_half(0, 1)
    half_pass(0, 0, True)

    def f_body(f, carry):
        start_half(f + 1, 0)
        half_pass(f, 1, False)
        start_half(f + 1, 1)
        half_pass(f + 1, 0, False)
        return carry

    lax.fori_loop(0, F - 1, f_body, 0)
    half_pass(F - 1, 1, False)

    def s_body(k, carry):
        sl = pl.ds(k * L, L)
        acc_v[sl] = acc_v[sl] * jnp.float32(1.0 / F)
        return carry

    lax.fori_loop(0, B // L, s_body, 0, unroll=8)
    pltpu.sync_copy(acc_v, out_hbm.at[d, :])


def kernel(c, tables):
    tt = tables.transpose(0, 2, 1)
    ct = c.T
    tails = tables[:, VH + VB:, :].transpose(0, 2, 1).reshape(-1)
    out_t = _emb_kernel(ct, tt, tails)
    return out_t.T


# speedup vs baseline: 1.2855x; 1.2855x over previous
import functools

import jax
import jax.numpy as jnp
from jax import lax
from jax.experimental import pallas as pl
from jax.experimental.pallas import tpu as pltpu
from jax.experimental.pallas import tpu_sc as plsc

F = 26
V = 100000
D = 32
B = 16384
L = 16

NC, NS = 2, 16
NW = NC * NS

CQ = 4096
NQ = B // CQ

_mesh = plsc.VectorSubcoreMesh(core_axis_name="c", subcore_axis_name="s")


@functools.partial(
    pl.kernel,
    mesh=_mesh,
    out_type=jax.ShapeDtypeStruct((D, B), jnp.float32),
    scratch_types=[
        pltpu.VMEM((V,), jnp.float32),
        pltpu.VMEM((CQ,), jnp.int32),
        pltpu.VMEM((CQ,), jnp.int32),
        pltpu.VMEM((B,), jnp.float32),
        pltpu.SemaphoreType.DMA,
        pltpu.SemaphoreType.DMA,
        pltpu.SemaphoreType.DMA,
    ],
    compiler_params=pltpu.CompilerParams(needs_layout_passes=False),
)
def _emb_kernel(ct_hbm, tt_hbm, out_hbm, row_v, cq0_v, cq1_v, acc_v,
                sem_r, sem_c0, sem_c1):
    d = lax.axis_index("s") * NC + lax.axis_index("c")
    bufs = (cq0_v, cq1_v)
    sems = (sem_c0, sem_c1)

    def field(f, store):
        row_cp = pltpu.async_copy(tt_hbm.at[f, d, :], row_v, sem_r)
        cps = [
            pltpu.async_copy(ct_hbm.at[f, pl.ds(0, CQ)], bufs[0], sems[0]),
            None,
        ]
        row_cp.wait()
        for q in range(NQ):
            b = q % 2
            cps[b].wait()
            if q + 1 < NQ:
                nb = (q + 1) % 2
                cps[nb] = pltpu.async_copy(
                    ct_hbm.at[f, pl.ds((q + 1) * CQ, CQ)], bufs[nb],
                    sems[nb])
            cq_v = bufs[b]

            def k_body(k, carry3):
                idx = cq_v[pl.ds(k * L, L)]
                g = plsc.load_gather(row_v, [idx])
                sl = pl.ds(q * CQ + k * L, L)
                if store:
                    acc_v[sl] = g
                else:
                    plsc.addupdate(acc_v.at[sl], g)
                return carry3

            lax.fori_loop(0, CQ // L, k_body, 0, unroll=16)

    field(0, True)

    def f_body(f, carry):
        field(f, False)
        return carry

    lax.fori_loop(1, F, f_body, 0)

    def s_body(k, carry):
        sl = pl.ds(k * L, L)
        acc_v[sl] = acc_v[sl] * jnp.float32(1.0 / F)
        return carry

    lax.fori_loop(0, B // L, s_body, 0, unroll=8)
    pltpu.sync_copy(acc_v, out_hbm.at[d, :])


def kernel(c, tables):
    tt = tables.transpose(0, 2, 1)
    ct = c.T
    out_t = _emb_kernel(ct, tt)
    return out_t.T
